# Initial kernel scaffold; baseline (speedup 1.0000x reference)
import jax
import jax.numpy as jnp
from jax.experimental import pallas as pl


def kernel(x, wave_segments, noise_added_flags, recon_noisy_flags, p_in_w, p_in_b, p_out_w, p_out_b, p_cb, c_in_w, c_in_b, c_out_w, c_out_b, c_cb, t_in_w, t_in_b, t_out_w, t_out_b, t_cb, r_in_w, r_in_b, r_out_w, r_out_b, r_cb):
    raise NotImplementedError("write your pallas kernel here")


# speedup vs baseline: 2.3441x; 2.3441x over previous
import functools

import jax
import jax.numpy as jnp
from jax.experimental import pallas as pl
from jax.experimental.pallas import tpu as pltpu

B = 16
T = 512
D = 1024
CB = 1024
CD = 8
NS = 8


def _mm(a, b, precision=None):
    return jax.lax.dot_general(a, b, (((1,), (0,)), ((), ())),
                               preferred_element_type=jnp.float32,
                               precision=precision)


def _fused_body(x_ref, w_in_ref, in_b_ref, out_w_ref, out_b_ref, cb_ref,
                mon_ref, moff_ref, mbase_ref,
                outs_ref, zp_ref, zc_ref, zt_ref, zr_ref, loss_ref,
                cbn_scr):
    b = pl.program_id(0)

    @pl.when(b == 0)
    def _prep():
        for s in range(NS):
            c = cb_ref[s]
            nrm = jnp.sqrt(jnp.sum(c * c, axis=1, keepdims=True))
            cbn_scr[s] = c / jnp.maximum(nrm, 1e-12)
        loss_ref[...] = jnp.zeros((1, 1), jnp.float32)

    xb = x_ref[0]
    loss = [loss_ref[...]]

    def stage(s, resid):
        e = (_mm(w_in_ref[pl.ds(CD * s, CD), :], resid)
             + in_b_ref[pl.ds(CD * s, CD), :])
        nrm2 = jnp.sum(e * e, axis=0, keepdims=True)
        en = e / jnp.maximum(jnp.sqrt(nrm2), 1e-12)
        se = jnp.sum(en * en, axis=0, keepdims=True)
        cbn = cbn_scr[s]
        cc = jnp.sum(cbn * cbn, axis=1, keepdims=True)
        dist = (se - 2.0 * _mm(cbn, en)) + cc
        iota = jax.lax.broadcasted_iota(jnp.int32, (CB, T), 0)
        dmin = jnp.min(dist, axis=0, keepdims=True)
        idx = jnp.min(jnp.where(dist == dmin, iota, CB), axis=0, keepdims=True)
        onehot = (iota == idx).astype(jnp.float32)
        q = jax.lax.dot_general(cb_ref[s], onehot, (((0,), (0,)), ((), ())),
                                preferred_element_type=jnp.float32,
                                precision=jax.lax.Precision.HIGHEST)
        loss[0] = loss[0] + jnp.sum((e - q) * (e - q), keepdims=True)
        q_st = e + (q - e)
        return _mm(out_w_ref[s], q_st) + out_b_ref[s]

    z_p = stage(0, xb)
    zc0 = stage(1, xb)
    zc1 = stage(2, xb - zc0)
    z_c = zc0 + zc1
    tr = (xb - z_p) - z_c
    zt0 = stage(3, tr)
    zt1 = stage(4, tr - zt0)
    z_t = zt0 + zt1
    rf = tr - z_t
    zr0 = stage(5, rf)
    zr1 = stage(6, (rf - zr0))
    zr2 = stage(7, (rf - zr0) - zr1)
    z_r = (zr0 + zr1) + zr2

    loss_ref[...] = loss[0]
    zp_ref[0] = z_p
    zc_ref[0] = z_c
    zt_ref[0] = z_t
    zr_ref[0] = z_r

    mask = jnp.where(mon_ref[0, :, :1] > 0.0, 1.0,
                     jnp.where(moff_ref[0, :, :1] > 0.0, 0.0,
                               mbase_ref[0, :, :1]))
    outs_ref[0] = ((z_p + z_c) + z_t) + z_r * mask


@functools.partial(jax.jit, static_argnames=("interpret",))
def _run(x, w_in, in_b_all, out_w_all, out_b_all, cb_all,
         m_on, m_off, m_base, interpret=False):
    grid = (B,)
    full = lambda *shape: pl.BlockSpec(shape, lambda b: (0,) * len(shape))
    per_b3 = pl.BlockSpec((1, D, T), lambda b: (b, 0, 0))
    per_b_s = pl.BlockSpec((1, 1, 128), lambda b: (b, 0, 0))
    outs = pl.pallas_call(
        _fused_body,
        grid=grid,
        in_specs=[
            per_b3,
            full(NS * CD, D),
            full(NS * CD, 1),
            full(NS, D, CD),
            full(NS, D, 1),
            full(NS, CB, CD),
            per_b_s, per_b_s, per_b_s,
        ],
        out_specs=[
            per_b3, per_b3, per_b3, per_b3, per_b3,
            pl.BlockSpec((1, 1), lambda b: (0, 0)),
        ],
        out_shape=[
            jax.ShapeDtypeStruct((B, D, T), jnp.float32),
            jax.ShapeDtypeStruct((B, D, T), jnp.float32),
            jax.ShapeDtypeStruct((B, D, T), jnp.float32),
            jax.ShapeDtypeStruct((B, D, T), jnp.float32),
            jax.ShapeDtypeStruct((B, D, T), jnp.float32),
            jax.ShapeDtypeStruct((1, 1), jnp.float32),
        ],
        scratch_shapes=[
            pltpu.VMEM((NS, CB, CD), jnp.float32),
        ],
        interpret=interpret,
    )(x, w_in, in_b_all, out_w_all, out_b_all, cb_all, m_on, m_off, m_base)
    return outs


def kernel(x, wave_segments, noise_added_flags, recon_noisy_flags,
           p_in_w, p_in_b, p_out_w, p_out_b, p_cb,
           c_in_w, c_in_b, c_out_w, c_out_b, c_cb,
           t_in_w, t_in_b, t_out_w, t_out_b, t_cb,
           r_in_w, r_in_b, r_out_w, r_out_b, r_cb,
           interpret=False):
    del wave_segments
    in_w = jnp.concatenate([p_in_w, c_in_w, t_in_w, r_in_w], axis=0)
    w_in = in_w.reshape(NS * CD, D)
    in_b_all = jnp.concatenate([p_in_b, c_in_b, t_in_b, r_in_b],
                               axis=0).reshape(NS * CD, 1)
    out_w_all = jnp.concatenate([p_out_w, c_out_w, t_out_w, r_out_w], axis=0)
    out_b_all = jnp.concatenate([p_out_b, c_out_b, t_out_b, r_out_b],
                                axis=0).reshape(NS, D, 1)
    cb_all = jnp.concatenate([p_cb, c_cb, t_cb, r_cb], axis=0)

    base = (jax.random.uniform(jax.random.key(42), (B,)) >= 0.75).astype(jnp.float32)
    m_on = (noise_added_flags & recon_noisy_flags).astype(jnp.float32)
    m_off = (noise_added_flags & (~recon_noisy_flags)).astype(jnp.float32)
    tile = lambda v: jnp.broadcast_to(v[:, None, None], (B, 1, 128))
    m_on, m_off, m_base = tile(m_on), tile(m_off), tile(base)

    outs, z_p, z_c, z_t, z_r, loss = _run(
        x, w_in, in_b_all, out_w_all, out_b_all, cb_all,
        m_on, m_off, m_base, interpret=interpret)
    lscalar = (loss / jnp.float32(B * CD * T)).reshape(())
    return (outs, z_p, z_c, z_t, z_r, lscalar, lscalar)


# speedup vs baseline: 3.6774x; 1.5688x over previous
import functools

import jax
import jax.numpy as jnp
from jax.experimental import pallas as pl
from jax.experimental.pallas import tpu as pltpu

B = 16
T = 512
D = 1024
CB = 1024
CD = 8
NS = 8


def _mm(a, b, precision=None):
    return jax.lax.dot_general(a, b, (((1,), (0,)), ((), ())),
                               preferred_element_type=jnp.float32,
                               precision=precision)


def _fused_body(x_ref, w_in_ref, in_b_ref, out_w_ref, out_b_ref, cb_ref,
                cbt_ref, mon_ref, moff_ref, mbase_ref,
                outs_ref, zp_ref, zc_ref, zt_ref, zr_ref, loss_ref,
                cbn2_scr, cc_scr):
    b = pl.program_id(0)

    @pl.when(b == 0)
    def _prep():
        for s in range(NS):
            c = cb_ref[s]
            nrm = jnp.sqrt(jnp.sum(c * c, axis=1, keepdims=True))
            cbn = c / jnp.maximum(nrm, 1e-12)
            cbn2_scr[s] = -2.0 * cbn
            cc_scr[s] = jnp.sum(cbn * cbn, axis=1, keepdims=True)
        loss_ref[...] = jnp.zeros((1, 1), jnp.float32)

    xb = x_ref[0]
    loss = [loss_ref[...]]

    def stage(s, resid):
        e = (_mm(w_in_ref[pl.ds(CD * s, CD), :], resid)
             + in_b_ref[pl.ds(CD * s, CD), :])
        nrm2 = jnp.sum(e * e, axis=0, keepdims=True)
        en = e / jnp.maximum(jnp.sqrt(nrm2), 1e-12)
        se = jnp.sum(en * en, axis=0, keepdims=True)
        cc = cc_scr[s]
        dist = (se + _mm(cbn2_scr[s], en)) + cc
        iota = jax.lax.broadcasted_iota(jnp.int32, (CB, T), 0)
        dmin = jnp.min(dist, axis=0, keepdims=True)
        idx = jnp.min(jnp.where(dist == dmin, iota, CB), axis=0, keepdims=True)
        idxb = jnp.broadcast_to(idx, (CD, T))
        low = jnp.bitwise_and(idxb, 127)
        hi = jax.lax.shift_right_logical(idxb, 7)
        q = None
        for k in range(CB // 128):
            gk = jnp.take_along_axis(cbt_ref[s][:, 128 * k:128 * (k + 1)],
                                     low, axis=1)
            q = gk if q is None else jnp.where(hi == k, gk, q)
        loss[0] = loss[0] + jnp.sum((e - q) * (e - q), keepdims=True)
        q_st = e + (q - e)
        return _mm(out_w_ref[s], q_st) + out_b_ref[s]

    z_p = stage(0, xb)
    zc0 = stage(1, xb)
    zc1 = stage(2, xb - zc0)
    z_c = zc0 + zc1
    tr = (xb - z_p) - z_c
    zt0 = stage(3, tr)
    zt1 = stage(4, tr - zt0)
    z_t = zt0 + zt1
    rf = tr - z_t
    zr0 = stage(5, rf)
    zr1 = stage(6, (rf - zr0))
    zr2 = stage(7, (rf - zr0) - zr1)
    z_r = (zr0 + zr1) + zr2

    loss_ref[...] = loss[0]
    zp_ref[0] = z_p
    zc_ref[0] = z_c
    zt_ref[0] = z_t
    zr_ref[0] = z_r

    mask = jnp.where(mon_ref[0, :, :1] > 0.0, 1.0,
                     jnp.where(moff_ref[0, :, :1] > 0.0, 0.0,
                               mbase_ref[0, :, :1]))
    outs_ref[0] = ((z_p + z_c) + z_t) + z_r * mask


@functools.partial(jax.jit, static_argnames=("interpret",))
def _run(x, w_in, in_b_all, out_w_all, out_b_all, cb_all, cbt_all,
         m_on, m_off, m_base, interpret=False):
    grid = (B,)
    full = lambda *shape: pl.BlockSpec(shape, lambda b: (0,) * len(shape))
    per_b3 = pl.BlockSpec((1, D, T), lambda b: (b, 0, 0))
    per_b_s = pl.BlockSpec((1, 1, 128), lambda b: (b, 0, 0))
    outs = pl.pallas_call(
        _fused_body,
        grid=grid,
        in_specs=[
            per_b3,
            full(NS * CD, D),
            full(NS * CD, 1),
            full(NS, D, CD),
            full(NS, D, 1),
            full(NS, CB, CD),
            full(NS, CD, CB),
            per_b_s, per_b_s, per_b_s,
        ],
        out_specs=[
            per_b3, per_b3, per_b3, per_b3, per_b3,
            pl.BlockSpec((1, 1), lambda b: (0, 0)),
        ],
        out_shape=[
            jax.ShapeDtypeStruct((B, D, T), jnp.float32),
            jax.ShapeDtypeStruct((B, D, T), jnp.float32),
            jax.ShapeDtypeStruct((B, D, T), jnp.float32),
            jax.ShapeDtypeStruct((B, D, T), jnp.float32),
            jax.ShapeDtypeStruct((B, D, T), jnp.float32),
            jax.ShapeDtypeStruct((1, 1), jnp.float32),
        ],
        scratch_shapes=[
            pltpu.VMEM((NS, CB, CD), jnp.float32),
            pltpu.VMEM((NS, CB, 1), jnp.float32),
        ],
        interpret=interpret,
    )(x, w_in, in_b_all, out_w_all, out_b_all, cb_all, cbt_all,
      m_on, m_off, m_base)
    return outs


def kernel(x, wave_segments, noise_added_flags, recon_noisy_flags,
           p_in_w, p_in_b, p_out_w, p_out_b, p_cb,
           c_in_w, c_in_b, c_out_w, c_out_b, c_cb,
           t_in_w, t_in_b, t_out_w, t_out_b, t_cb,
           r_in_w, r_in_b, r_out_w, r_out_b, r_cb,
           interpret=False):
    del wave_segments
    in_w = jnp.concatenate([p_in_w, c_in_w, t_in_w, r_in_w], axis=0)
    w_in = in_w.reshape(NS * CD, D)
    in_b_all = jnp.concatenate([p_in_b, c_in_b, t_in_b, r_in_b],
                               axis=0).reshape(NS * CD, 1)
    out_w_all = jnp.concatenate([p_out_w, c_out_w, t_out_w, r_out_w], axis=0)
    out_b_all = jnp.concatenate([p_out_b, c_out_b, t_out_b, r_out_b],
                                axis=0).reshape(NS, D, 1)
    cb_all = jnp.concatenate([p_cb, c_cb, t_cb, r_cb], axis=0)
    cbt_all = cb_all.transpose(0, 2, 1)

    base = (jax.random.uniform(jax.random.key(42), (B,)) >= 0.75).astype(jnp.float32)
    m_on = (noise_added_flags & recon_noisy_flags).astype(jnp.float32)
    m_off = (noise_added_flags & (~recon_noisy_flags)).astype(jnp.float32)
    tile = lambda v: jnp.broadcast_to(v[:, None, None], (B, 1, 128))
    m_on, m_off, m_base = tile(m_on), tile(m_off), tile(base)

    outs, z_p, z_c, z_t, z_r, loss = _run(
        x, w_in, in_b_all, out_w_all, out_b_all, cb_all, cbt_all,
        m_on, m_off, m_base, interpret=interpret)
    lscalar = (loss / jnp.float32(B * CD * T)).reshape(())
    return (outs, z_p, z_c, z_t, z_r, lscalar, lscalar)


# speedup vs baseline: 4.2857x; 1.1654x over previous
import functools

import jax
import jax.numpy as jnp
from jax.experimental import pallas as pl
from jax.experimental.pallas import tpu as pltpu

B = 16
T = 512
D = 1024
CB = 1024
CD = 8
NS = 8


def _mm(a, b, precision=None):
    return jax.lax.dot_general(a, b, (((1,), (0,)), ((), ())),
                               preferred_element_type=jnp.float32,
                               precision=precision)


def _fused_body(x_ref, w_in_ref, in_b_ref, out_w_ref, out_b_ref, cb_ref,
                cbt_ref, mon_ref, moff_ref, mbase_ref,
                outs_ref, zp_ref, zc_ref, zt_ref, zr_ref, loss_ref,
                cbn2_scr, cc_scr):
    b = pl.program_id(0)

    @pl.when(b == 0)
    def _prep():
        for s in range(NS):
            c = cb_ref[s]
            nrm = jnp.sqrt(jnp.sum(c * c, axis=1, keepdims=True))
            cbn = c / jnp.maximum(nrm, 1e-12)
            cbn2_scr[s] = -2.0 * cbn
            cc_scr[s] = jnp.sum(cbn * cbn, axis=1, keepdims=True)
        loss_ref[...] = jnp.zeros((1, 1), jnp.float32)

    xb = x_ref[0]
    loss = [loss_ref[...]]

    def stage(s, resid):
        e = (_mm(w_in_ref[pl.ds(CD * s, CD), :], resid)
             + in_b_ref[pl.ds(CD * s, CD), :])
        nrm2 = jnp.sum(e * e, axis=0, keepdims=True)
        en = e / jnp.maximum(jnp.sqrt(nrm2), 1e-12)
        se = jnp.sum(en * en, axis=0, keepdims=True)
        cc = cc_scr[s]
        dist = (se + _mm(cbn2_scr[s], en)) + cc
        idx = jnp.argmin(dist, axis=0)[None, :]
        idxb = jnp.broadcast_to(idx, (CD, T))
        low = jnp.bitwise_and(idxb, 127)
        hi = jax.lax.shift_right_logical(idxb, 7)
        q = None
        for k in range(CB // 128):
            gk = jnp.take_along_axis(cbt_ref[s][:, 128 * k:128 * (k + 1)],
                                     low, axis=1)
            q = gk if q is None else jnp.where(hi == k, gk, q)
        loss[0] = loss[0] + jnp.sum((e - q) * (e - q), keepdims=True)
        q_st = e + (q - e)
        return _mm(out_w_ref[s], q_st) + out_b_ref[s]

    z_p = stage(0, xb)
    zc0 = stage(1, xb)
    zc1 = stage(2, xb - zc0)
    z_c = zc0 + zc1
    tr = (xb - z_p) - z_c
    zt0 = stage(3, tr)
    zt1 = stage(4, tr - zt0)
    z_t = zt0 + zt1
    rf = tr - z_t
    zr0 = stage(5, rf)
    zr1 = stage(6, (rf - zr0))
    zr2 = stage(7, (rf - zr0) - zr1)
    z_r = (zr0 + zr1) + zr2

    loss_ref[...] = loss[0]
    zp_ref[0] = z_p
    zc_ref[0] = z_c
    zt_ref[0] = z_t
    zr_ref[0] = z_r

    mask = jnp.where(mon_ref[0, :, :1] > 0.0, 1.0,
                     jnp.where(moff_ref[0, :, :1] > 0.0, 0.0,
                               mbase_ref[0, :, :1]))
    outs_ref[0] = ((z_p + z_c) + z_t) + z_r * mask


@functools.partial(jax.jit, static_argnames=("interpret",))
def _run(x, w_in, in_b_all, out_w_all, out_b_all, cb_all, cbt_all,
         m_on, m_off, m_base, interpret=False):
    grid = (B,)
    full = lambda *shape: pl.BlockSpec(shape, lambda b: (0,) * len(shape))
    per_b3 = pl.BlockSpec((1, D, T), lambda b: (b, 0, 0))
    per_b_s = pl.BlockSpec((1, 1, 128), lambda b: (b, 0, 0))
    outs = pl.pallas_call(
        _fused_body,
        grid=grid,
        in_specs=[
            per_b3,
            full(NS * CD, D),
            full(NS * CD, 1),
            full(NS, D, CD),
            full(NS, D, 1),
            full(NS, CB, CD),
            full(NS, CD, CB),
            per_b_s, per_b_s, per_b_s,
        ],
        out_specs=[
            per_b3, per_b3, per_b3, per_b3, per_b3,
            pl.BlockSpec((1, 1), lambda b: (0, 0)),
        ],
        out_shape=[
            jax.ShapeDtypeStruct((B, D, T), jnp.float32),
            jax.ShapeDtypeStruct((B, D, T), jnp.float32),
            jax.ShapeDtypeStruct((B, D, T), jnp.float32),
            jax.ShapeDtypeStruct((B, D, T), jnp.float32),
            jax.ShapeDtypeStruct((B, D, T), jnp.float32),
            jax.ShapeDtypeStruct((1, 1), jnp.float32),
        ],
        scratch_shapes=[
            pltpu.VMEM((NS, CB, CD), jnp.float32),
            pltpu.VMEM((NS, CB, 1), jnp.float32),
        ],
        interpret=interpret,
    )(x, w_in, in_b_all, out_w_all, out_b_all, cb_all, cbt_all,
      m_on, m_off, m_base)
    return outs


def kernel(x, wave_segments, noise_added_flags, recon_noisy_flags,
           p_in_w, p_in_b, p_out_w, p_out_b, p_cb,
           c_in_w, c_in_b, c_out_w, c_out_b, c_cb,
           t_in_w, t_in_b, t_out_w, t_out_b, t_cb,
           r_in_w, r_in_b, r_out_w, r_out_b, r_cb,
           interpret=False):
    del wave_segments
    in_w = jnp.concatenate([p_in_w, c_in_w, t_in_w, r_in_w], axis=0)
    w_in = in_w.reshape(NS * CD, D)
    in_b_all = jnp.concatenate([p_in_b, c_in_b, t_in_b, r_in_b],
                               axis=0).reshape(NS * CD, 1)
    out_w_all = jnp.concatenate([p_out_w, c_out_w, t_out_w, r_out_w], axis=0)
    out_b_all = jnp.concatenate([p_out_b, c_out_b, t_out_b, r_out_b],
                                axis=0).reshape(NS, D, 1)
    cb_all = jnp.concatenate([p_cb, c_cb, t_cb, r_cb], axis=0)
    cbt_all = cb_all.transpose(0, 2, 1)

    base = (jax.random.uniform(jax.random.key(42), (B,)) >= 0.75).astype(jnp.float32)
    m_on = (noise_added_flags & recon_noisy_flags).astype(jnp.float32)
    m_off = (noise_added_flags & (~recon_noisy_flags)).astype(jnp.float32)
    tile = lambda v: jnp.broadcast_to(v[:, None, None], (B, 1, 128))
    m_on, m_off, m_base = tile(m_on), tile(m_off), tile(base)

    outs, z_p, z_c, z_t, z_r, loss = _run(
        x, w_in, in_b_all, out_w_all, out_b_all, cb_all, cbt_all,
        m_on, m_off, m_base, interpret=interpret)
    lscalar = (loss / jnp.float32(B * CD * T)).reshape(())
    return (outs, z_p, z_c, z_t, z_r, lscalar, lscalar)


# speedup vs baseline: 4.4138x; 1.0299x over previous
import functools

import jax
import jax.numpy as jnp
from jax.experimental import pallas as pl
from jax.experimental.pallas import tpu as pltpu

B = 16
T = 512
D = 1024
CB = 1024
CD = 8
NS = 8


def _mm(a, b, precision=None):
    return jax.lax.dot_general(a, b, (((1,), (0,)), ((), ())),
                               preferred_element_type=jnp.float32,
                               precision=precision)


def _fused_body(x_ref, w_in_ref, in_b_ref, out_w_ref, out_b_ref, cb_ref,
                cbt_ref, mon_ref, moff_ref, mbase_ref,
                outs_ref, zp_ref, zc_ref, zt_ref, zr_ref, loss_ref,
                cbn2_scr, cc_scr):
    b = pl.program_id(0)

    @pl.when(b == 0)
    def _prep():
        for s in range(NS):
            c = cb_ref[s]
            nrm = jnp.sqrt(jnp.sum(c * c, axis=1, keepdims=True))
            cbn = c / jnp.maximum(nrm, 1e-12)
            cbn2_scr[s] = -2.0 * cbn
            cc_scr[s] = jnp.sum(cbn * cbn, axis=1, keepdims=True)
        loss_ref[...] = jnp.zeros((1, 1), jnp.float32)

    xb = x_ref[0]
    loss = [loss_ref[...]]

    def stage(s, resid, e=None):
        if e is None:
            e = (_mm(w_in_ref[pl.ds(CD * s, CD), :], resid)
                 + in_b_ref[pl.ds(CD * s, CD), :])
        nrm2 = jnp.sum(e * e, axis=0, keepdims=True)
        en = e / jnp.maximum(jnp.sqrt(nrm2), 1e-12)
        se = jnp.sum(en * en, axis=0, keepdims=True)
        cc = cc_scr[s]
        dist = (se + _mm(cbn2_scr[s], en)) + cc
        idx = jnp.argmin(dist, axis=0)[None, :]
        idxb = jnp.broadcast_to(idx, (CD, T))
        low = jnp.bitwise_and(idxb, 127)
        hi = jax.lax.shift_right_logical(idxb, 7)
        q = None
        for k in range(CB // 128):
            gk = jnp.take_along_axis(cbt_ref[s][:, 128 * k:128 * (k + 1)],
                                     low, axis=1)
            q = gk if q is None else jnp.where(hi == k, gk, q)
        loss[0] = loss[0] + jnp.sum((e - q) * (e - q), keepdims=True)
        q_st = e + (q - e)
        return _mm(out_w_ref[s], q_st) + out_b_ref[s]

    e01 = _mm(w_in_ref[pl.ds(0, 2 * CD), :], xb)
    z_p = stage(0, xb, e=e01[0:CD, :] + in_b_ref[pl.ds(0, CD), :])
    zc0 = stage(1, xb, e=e01[CD:2 * CD, :] + in_b_ref[pl.ds(CD, CD), :])
    zc1 = stage(2, xb - zc0)
    z_c = zc0 + zc1
    tr = (xb - z_p) - z_c
    zt0 = stage(3, tr)
    zt1 = stage(4, tr - zt0)
    z_t = zt0 + zt1
    rf = tr - z_t
    zr0 = stage(5, rf)
    zr1 = stage(6, (rf - zr0))
    zr2 = stage(7, (rf - zr0) - zr1)
    z_r = (zr0 + zr1) + zr2

    loss_ref[...] = loss[0]
    zp_ref[0] = z_p
    zc_ref[0] = z_c
    zt_ref[0] = z_t
    zr_ref[0] = z_r

    mask = jnp.where(mon_ref[0, :, :1] > 0.0, 1.0,
                     jnp.where(moff_ref[0, :, :1] > 0.0, 0.0,
                               mbase_ref[0, :, :1]))
    outs_ref[0] = (xb - rf) + z_r * mask


@functools.partial(jax.jit, static_argnames=("interpret",))
def _run(x, w_in, in_b_all, out_w_all, out_b_all, cb_all, cbt_all,
         m_on, m_off, m_base, interpret=False):
    grid = (B,)
    full = lambda *shape: pl.BlockSpec(shape, lambda b: (0,) * len(shape))
    per_b3 = pl.BlockSpec((1, D, T), lambda b: (b, 0, 0))
    per_b_s = pl.BlockSpec((1, 1, 128), lambda b: (b, 0, 0))
    outs = pl.pallas_call(
        _fused_body,
        grid=grid,
        in_specs=[
            per_b3,
            full(NS * CD, D),
            full(NS * CD, 1),
            full(NS, D, CD),
            full(NS, D, 1),
            full(NS, CB, CD),
            full(NS, CD, CB),
            per_b_s, per_b_s, per_b_s,
        ],
        out_specs=[
            per_b3, per_b3, per_b3, per_b3, per_b3,
            pl.BlockSpec((1, 1), lambda b: (0, 0)),
        ],
        out_shape=[
            jax.ShapeDtypeStruct((B, D, T), jnp.float32),
            jax.ShapeDtypeStruct((B, D, T), jnp.float32),
            jax.ShapeDtypeStruct((B, D, T), jnp.float32),
            jax.ShapeDtypeStruct((B, D, T), jnp.float32),
            jax.ShapeDtypeStruct((B, D, T), jnp.float32),
            jax.ShapeDtypeStruct((1, 1), jnp.float32),
        ],
        scratch_shapes=[
            pltpu.VMEM((NS, CB, CD), jnp.float32),
            pltpu.VMEM((NS, CB, 1), jnp.float32),
        ],
        interpret=interpret,
    )(x, w_in, in_b_all, out_w_all, out_b_all, cb_all, cbt_all,
      m_on, m_off, m_base)
    return outs


def kernel(x, wave_segments, noise_added_flags, recon_noisy_flags,
           p_in_w, p_in_b, p_out_w, p_out_b, p_cb,
           c_in_w, c_in_b, c_out_w, c_out_b, c_cb,
           t_in_w, t_in_b, t_out_w, t_out_b, t_cb,
           r_in_w, r_in_b, r_out_w, r_out_b, r_cb,
           interpret=False):
    del wave_segments
    in_w = jnp.concatenate([p_in_w, c_in_w, t_in_w, r_in_w], axis=0)
    w_in = in_w.reshape(NS * CD, D)
    in_b_all = jnp.concatenate([p_in_b, c_in_b, t_in_b, r_in_b],
                               axis=0).reshape(NS * CD, 1)
    out_w_all = jnp.concatenate([p_out_w, c_out_w, t_out_w, r_out_w], axis=0)
    out_b_all = jnp.concatenate([p_out_b, c_out_b, t_out_b, r_out_b],
                                axis=0).reshape(NS, D, 1)
    cb_all = jnp.concatenate([p_cb, c_cb, t_cb, r_cb], axis=0)
    cbt_all = cb_all.transpose(0, 2, 1)

    base = (jax.random.uniform(jax.random.key(42), (B,)) >= 0.75).astype(jnp.float32)
    m_on = (noise_added_flags & recon_noisy_flags).astype(jnp.float32)
    m_off = (noise_added_flags & (~recon_noisy_flags)).astype(jnp.float32)
    tile = lambda v: jnp.broadcast_to(v[:, None, None], (B, 1, 128))
    m_on, m_off, m_base = tile(m_on), tile(m_off), tile(base)

    outs, z_p, z_c, z_t, z_r, loss = _run(
        x, w_in, in_b_all, out_w_all, out_b_all, cb_all, cbt_all,
        m_on, m_off, m_base, interpret=interpret)
    lscalar = (loss / jnp.float32(B * CD * T)).reshape(())
    return (outs, z_p, z_c, z_t, z_r, lscalar, lscalar)
